# baseline (speedup 1.0000x reference)
import functools

import jax
import jax.numpy as jnp
from jax import lax
from jax.experimental import pallas as pl
from jax.experimental.pallas import tpu as pltpu
from jax.experimental.pallas import tpu_sc as plsc

_NBARS = 100
_ROWS = 101
_NW = 32
_TPW = 1024
_CHUNK = 256
_L = 16
_LN2 = 0.6931471805599453


_RB = 2048


def _tc_body(x_ref, a_ref):
    x = x_ref[...]
    m = jnp.max(x, axis=1, keepdims=True)
    a_ref[0:_NBARS, :] = x.T
    a_ref[_NBARS:_ROWS, :] = m.T


def _tc_prep(logits2):
    n = logits2.shape[0]
    return pl.pallas_call(
        _tc_body,
        grid=(n // _RB,),
        in_specs=[pl.BlockSpec((_RB, _NBARS), lambda i: (i, 0))],
        out_specs=pl.BlockSpec((_ROWS, _RB), lambda i: (0, i)),
        out_shape=jax.ShapeDtypeStruct((_ROWS, n), jnp.float32),
    )(logits2)


def _ln(x):
    bits = lax.bitcast_convert_type(x, jnp.int32)
    e = ((bits >> 23) & 255) - 127
    m = lax.bitcast_convert_type((bits & 0x007FFFFF) | 0x3F800000, jnp.float32)
    big = m > 1.4142135
    m = jnp.where(big, m * 0.5, m)
    ef = (e + jnp.where(big, 1, 0)).astype(jnp.float32)
    t = (m - 1.0) / (m + 1.0)
    t2 = t * t
    p = 1.0 + t2 * (0.3333333333 + t2 * (0.2 + t2 * (0.1428571429 + t2 * 0.1111111111)))
    return ef * _LN2 + (2.0 * t) * p


def _bc(v):
    return jnp.full((_L,), v, jnp.int32)


def _sc_body(a_hbm, y_hbm, borders_hbm, out_hbm,
             buf0, buf1, y_v, out_v, borders_v, sem0, sem1):
    wid = lax.axis_index("s") * 2 + lax.axis_index("c")
    tok0 = wid * _TPW
    yrow0 = wid * 8
    bufs = (buf0, buf1)
    sems = (sem0, sem1)
    cps = [pltpu.async_copy(a_hbm.at[:, pl.ds(tok0, _CHUNK)], buf0, sem0)]
    pltpu.sync_copy(borders_hbm, borders_v)
    pltpu.sync_copy(y_hbm.at[pl.ds(yrow0, 8)], y_v)
    lanes = lax.iota(jnp.int32, _L)
    nchunks = _TPW // _CHUNK

    for cc in range(nchunks):
        buf = bufs[cc % 2]
        if cc + 1 < nchunks:
            cps.append(pltpu.async_copy(
                a_hbm.at[:, pl.ds(tok0 + (cc + 1) * _CHUNK, _CHUNK)],
                bufs[(cc + 1) % 2], sems[(cc + 1) % 2]))
        cps[cc].wait()

        def group(g, carry):
            t0 = g * _L
            m = buf[_NBARS, pl.ds(t0, _L)]

            def pbody(ci, accs):
                c0 = ci * 10
                xs = [buf[c0 + j, pl.ds(t0, _L)] for j in range(10)]
                a = list(accs)
                for j in range(10):
                    a[j % 4] = a[j % 4] + jnp.exp(xs[j] - m)
                return tuple(a)

            z = jnp.zeros((_L,), jnp.float32)
            s4 = lax.fori_loop(0, 10, pbody, (z, z, z, z))
            s = (s4[0] + s4[1]) + (s4[2] + s4[3])
            lse = m + _ln(s)

            gg = cc * (_CHUNK // _L) + g
            yrow = _bc(gg // 8)
            ycol = (gg % 8) * _L + lanes
            yv = plsc.load_gather(y_v, [yrow, ycol])
            idx = jnp.clip((yv * float(_NBARS)).astype(jnp.int32), 0, _NBARS - 1)
            for _ in range(2):
                blo = plsc.load_gather(borders_v, [idx])
                bhi = plsc.load_gather(borders_v, [idx + 1])
                idx = idx - jnp.where(yv <= blo, 1, 0) + jnp.where(yv > bhi, 1, 0)
                idx = jnp.clip(idx, 0, _NBARS - 1)
            blo = plsc.load_gather(borders_v, [idx])
            bhi = plsc.load_gather(borders_v, [idx + 1])
            gl = plsc.load_gather(buf, [idx, t0 + lanes])
            nll = lse - gl + _ln(bhi - blo)
            plsc.store_scatter(out_v, [yrow, ycol], nll)
            return carry

        lax.fori_loop(0, _CHUNK // _L, group, 0)

    pltpu.sync_copy(out_v, out_hbm.at[pl.ds(yrow0, 8)])


@functools.partial(
    pl.kernel,
    mesh=plsc.VectorSubcoreMesh(core_axis_name="c", subcore_axis_name="s"),
    compiler_params=pltpu.CompilerParams(
        needs_layout_passes=False, use_tc_tiling_on_sc=True),
    out_type=jax.ShapeDtypeStruct((_NW * _TPW // 128, 128), jnp.float32),
    scratch_types=[
        pltpu.VMEM((_ROWS, _CHUNK), jnp.float32),
        pltpu.VMEM((_ROWS, _CHUNK), jnp.float32),
        pltpu.VMEM((8, 128), jnp.float32),
        pltpu.VMEM((8, 128), jnp.float32),
        pltpu.VMEM((_NBARS + 1,), jnp.float32),
        pltpu.SemaphoreType.DMA,
        pltpu.SemaphoreType.DMA,
    ],
)
def _sc_nll(a_hbm, y_hbm, borders_hbm, out_hbm,
            buf0, buf1, y_v, out_v, borders_v, sem0, sem1):
    _sc_body(a_hbm, y_hbm, borders_hbm, out_hbm,
             buf0, buf1, y_v, out_v, borders_v, sem0, sem1)


def kernel(logits, y, borders):
    b, t, nb = logits.shape
    n = b * t
    logits2 = logits.reshape(n, nb)
    a = _tc_prep(logits2)
    y2 = y.reshape(n // 128, 128)
    out = _sc_nll(a, y2, borders)
    return out.reshape(b, t)


# speedup vs baseline: 1.0221x; 1.0221x over previous
import functools

import jax
import jax.numpy as jnp
from jax import lax
from jax.experimental import pallas as pl
from jax.experimental.pallas import tpu as pltpu
from jax.experimental.pallas import tpu_sc as plsc

_NBARS = 100
_ROWS = 101
_NW = 32
_TPW = 1024
_CHUNK = 256
_L = 16
_LN2 = 0.6931471805599453

_B = 4
_T = 8192


_RB = 2048


def _tc_body(x_ref, a_ref):
    x = x_ref[...]
    m = jnp.max(x, axis=1, keepdims=True)
    a_ref[0:_NBARS, :] = x.T
    a_ref[_NBARS:_ROWS, :] = m.T


def _tc_prep(logits2):
    n = logits2.shape[0]
    return pl.pallas_call(
        _tc_body,
        grid=(n // _RB,),
        in_specs=[pl.BlockSpec((_RB, _NBARS), lambda i: (i, 0))],
        out_specs=pl.BlockSpec((_ROWS, _RB), lambda i: (0, i)),
        out_shape=jax.ShapeDtypeStruct((_ROWS, n), jnp.float32),
    )(logits2)


def _ln(x):
    bits = lax.bitcast_convert_type(x, jnp.int32)
    e = ((bits >> 23) & 255) - 127
    m = lax.bitcast_convert_type((bits & 0x007FFFFF) | 0x3F800000, jnp.float32)
    big = m > 1.4142135
    m = jnp.where(big, m * 0.5, m)
    ef = (e + jnp.where(big, 1, 0)).astype(jnp.float32)
    t = (m - 1.0) / (m + 1.0)
    t2 = t * t
    p = 1.0 + t2 * (0.3333333333 + t2 * (0.2 + t2 * (0.1428571429 + t2 * 0.1111111111)))
    return ef * _LN2 + (2.0 * t) * p


def _sc_body(a_hbm, y_hbm, borders_hbm, out_hbm,
             buf0, buf1, y_v, out_v, borders_v, sem0, sem1):
    wid = lax.axis_index("s") * 2 + lax.axis_index("c")
    tok0 = wid * _TPW
    bidx = wid // (_T // _TPW)
    t0b = (wid % (_T // _TPW)) * _TPW
    bufs = (buf0, buf1)
    sems = (sem0, sem1)
    cps = [pltpu.async_copy(a_hbm.at[:, pl.ds(tok0, _CHUNK)], buf0, sem0)]
    pltpu.sync_copy(borders_hbm, borders_v)
    pltpu.sync_copy(y_hbm.at[bidx, pl.ds(t0b, _TPW)], y_v)
    lanes = lax.iota(jnp.int32, _L)
    nchunks = _TPW // _CHUNK

    for cc in range(nchunks):
        buf = bufs[cc % 2]
        if cc + 1 < nchunks:
            cps.append(pltpu.async_copy(
                a_hbm.at[:, pl.ds(tok0 + (cc + 1) * _CHUNK, _CHUNK)],
                bufs[(cc + 1) % 2], sems[(cc + 1) % 2]))
        cps[cc].wait()

        def group(g, carry):
            t0 = g * _L
            m = buf[_NBARS, pl.ds(t0, _L)]

            def pbody(ci, accs):
                c0 = ci * 10
                xs = [buf[c0 + j, pl.ds(t0, _L)] for j in range(10)]
                a = list(accs)
                for j in range(10):
                    a[j % 4] = a[j % 4] + jnp.exp(xs[j] - m)
                return tuple(a)

            z = jnp.zeros((_L,), jnp.float32)
            s4 = lax.fori_loop(0, 10, pbody, (z, z, z, z))
            s = (s4[0] + s4[1]) + (s4[2] + s4[3])
            lse = m + _ln(s)

            gtok = cc * _CHUNK + t0 + lanes
            yv = plsc.load_gather(y_v, [gtok])
            idx = jnp.clip((yv * float(_NBARS)).astype(jnp.int32), 0, _NBARS - 1)
            for _ in range(2):
                blo = plsc.load_gather(borders_v, [idx])
                bhi = plsc.load_gather(borders_v, [idx + 1])
                idx = idx - jnp.where(yv <= blo, 1, 0) + jnp.where(yv > bhi, 1, 0)
                idx = jnp.clip(idx, 0, _NBARS - 1)
            blo = plsc.load_gather(borders_v, [idx])
            bhi = plsc.load_gather(borders_v, [idx + 1])
            gl = plsc.load_gather(buf, [idx, t0 + lanes])
            nll = lse - gl + _ln(bhi - blo)
            plsc.store_scatter(out_v, [gtok], nll)
            return carry

        lax.fori_loop(0, _CHUNK // _L, group, 0)

    pltpu.sync_copy(out_v, out_hbm.at[bidx, pl.ds(t0b, _TPW)])


@functools.partial(
    pl.kernel,
    mesh=plsc.VectorSubcoreMesh(core_axis_name="c", subcore_axis_name="s"),
    compiler_params=pltpu.CompilerParams(needs_layout_passes=False),
    out_type=jax.ShapeDtypeStruct((_B, _T), jnp.float32),
    scratch_types=[
        pltpu.VMEM((_ROWS, _CHUNK), jnp.float32),
        pltpu.VMEM((_ROWS, _CHUNK), jnp.float32),
        pltpu.VMEM((_TPW,), jnp.float32),
        pltpu.VMEM((_TPW,), jnp.float32),
        pltpu.VMEM((_NBARS + 1,), jnp.float32),
        pltpu.SemaphoreType.DMA,
        pltpu.SemaphoreType.DMA,
    ],
)
def _sc_nll(a_hbm, y_hbm, borders_hbm, out_hbm,
            buf0, buf1, y_v, out_v, borders_v, sem0, sem1):
    _sc_body(a_hbm, y_hbm, borders_hbm, out_hbm,
             buf0, buf1, y_v, out_v, borders_v, sem0, sem1)


def kernel(logits, y, borders):
    b, t, nb = logits.shape
    n = b * t
    a = _tc_prep(logits.reshape(n, nb))
    return _sc_nll(a, y, borders)
